# Initial kernel scaffold; baseline (speedup 1.0000x reference)
import jax
import jax.numpy as jnp
from jax.experimental import pallas as pl


def kernel(g, h, W0, b0, W1, b1, W2, b2, W3, b3, W4, b4, W5, b5):
    raise NotImplementedError("write your pallas kernel here")


# speedup vs baseline: 6.3598x; 6.3598x over previous
import numpy as np
import jax
import jax.numpy as jnp
from jax import lax
from jax.experimental import pallas as pl
from jax.experimental.pallas import tpu as pltpu

_N = 2048
_DIM = 256
_KS = [0.9, 0.8, 0.7, 0.6, 0.5, 0.4]
_KVALS = [max(2, int(kf * _N)) for kf in _KS]
_NLEV = 6
_LANES = 128


def _tc_body(h_ref, w_ref, b_ref, k_ref, out_ref):
    h = h_ref[...]
    wt = jnp.dot(h, w_ref[...], preferred_element_type=jnp.float32) + b_ref[...]
    ibits = lax.bitcast_convert_type(wt, jnp.int32)
    key = jnp.where(ibits >= 0, ibits, ibits ^ jnp.int32(0x7FFFFFFF))
    kvec = k_ref[...]

    def step(i, prefix):
        bit = 31 - i
        cand = prefix + (jnp.int32(1) << bit)
        cnt = jnp.sum((key >= cand).astype(jnp.int32), axis=0, keepdims=True)
        return jnp.where(cnt >= kvec, cand, prefix)

    prefix0 = jnp.full((1, _LANES), jnp.int32(-(2**31)), jnp.int32)
    thr = lax.fori_loop(0, 32, step, prefix0)
    mask = key >= thr
    col = lax.broadcasted_iota(jnp.int32, (1, _LANES), 1)
    valid = col < _NLEV
    scores = 1.0 / (1.0 + jnp.exp(-wt))
    scale = jnp.sum(jnp.where(mask & valid, scores, 0.0), axis=1, keepdims=True)
    out_ref[...] = h * scale


def _run_tc(h, Wp, bp, kv, interpret=False):
    return pl.pallas_call(
        _tc_body,
        out_shape=jax.ShapeDtypeStruct((_N, _DIM), jnp.float32),
        interpret=interpret,
    )(h, Wp, bp, kv)


_KV_CONST = np.full((1, _LANES), _N + 1, np.int32)
_KV_CONST[0, :_NLEV] = _KVALS


def kernel(g, h, W0, b0, W1, b1, W2, b2, W3, b3, W4, b4, W5, b5):
    del g
    Ws = jnp.concatenate([W0, W1, W2, W3, W4, W5], axis=1)
    Wp = jnp.zeros((_DIM, _LANES), jnp.float32).at[:, :_NLEV].set(Ws)
    bs = jnp.stack([b0[0], b1[0], b2[0], b3[0], b4[0], b5[0]])
    bp = jnp.zeros((1, _LANES), jnp.float32).at[0, :_NLEV].set(bs)
    kv = jnp.asarray(_KV_CONST)
    return _run_tc(h, Wp, bp, kv)


# speedup vs baseline: 8.3043x; 1.3058x over previous
import numpy as np
import jax
import jax.numpy as jnp
from jax import lax
from jax.experimental import pallas as pl
from jax.experimental.pallas import tpu as pltpu

_N = 2048
_DIM = 256
_KS = [0.9, 0.8, 0.7, 0.6, 0.5, 0.4]
_KVALS = [max(2, int(kf * _N)) for kf in _KS]
_NLEV = 6
_LEVPAD = 8


def _tc_body(h_ref, wT_ref, b_ref, k_ref, ones_ref, out_ref):
    h = h_ref[...]
    wtT = lax.dot_general(
        wT_ref[...], h, (((1,), (1,)), ((), ())),
        preferred_element_type=jnp.float32) + b_ref[...]
    ibits = lax.bitcast_convert_type(wtT, jnp.int32)
    key = jnp.where(ibits >= 0, ibits, ibits ^ jnp.int32(0x7FFFFFFF))
    kvec = k_ref[...]

    def step(i, prefix):
        bit = 31 - i
        cand = prefix + (jnp.int32(1) << bit)
        cnt = jnp.sum((key >= cand).astype(jnp.int32), axis=1, keepdims=True)
        return jnp.where(cnt >= kvec, cand, prefix)

    prefix0 = jnp.full((_LEVPAD, 1), jnp.int32(-(2**31)), jnp.int32)
    thr = lax.fori_loop(0, 32, step, prefix0)
    mask = key >= thr
    lev = lax.broadcasted_iota(jnp.int32, (_LEVPAD, 1), 0)
    valid = lev < _NLEV
    scores = 1.0 / (1.0 + jnp.exp(-wtT))
    contrib = jnp.where(mask & valid, scores, 0.0)
    scale = lax.dot_general(
        contrib, ones_ref[...], (((0,), (0,)), ((), ())),
        preferred_element_type=jnp.float32)
    out_ref[...] = h * scale


def _run_tc(h, WT8, b8, kv, ones8, interpret=False):
    return pl.pallas_call(
        _tc_body,
        out_shape=jax.ShapeDtypeStruct((_N, _DIM), jnp.float32),
        interpret=interpret,
    )(h, WT8, b8, kv, ones8)


_KV_CONST = np.full((_LEVPAD, 1), _N + 1, np.int32)
_KV_CONST[:_NLEV, 0] = _KVALS


def kernel(g, h, W0, b0, W1, b1, W2, b2, W3, b3, W4, b4, W5, b5):
    del g
    Ws = jnp.concatenate([W0, W1, W2, W3, W4, W5], axis=1)
    WT8 = jnp.zeros((_LEVPAD, _DIM), jnp.float32).at[:_NLEV, :].set(Ws.T)
    bs = jnp.stack([b0[0], b1[0], b2[0], b3[0], b4[0], b5[0]])
    b8 = jnp.zeros((_LEVPAD, 1), jnp.float32).at[:_NLEV, 0].set(bs)
    kv = jnp.asarray(_KV_CONST)
    ones8 = jnp.ones((_LEVPAD, 1), jnp.float32)
    return _run_tc(h, WT8, b8, kv, ones8)
